# baseline (speedup 1.0000x reference)
import functools

import jax
import jax.numpy as jnp
import numpy as np
from jax import lax
from jax.experimental import pallas as pl
from jax.experimental.pallas import tpu as pltpu
from jax.experimental.pallas import tpu_sc as plsc

_N = 16777216
_L = 16
_NW = 32
_CH = 16384
_K = 7340032
_PER_W = _K // _NW
_CHUNKS = _PER_W // _CH
_ROWS = _N // 1024
_K_ROWS = _K // 1024
_BM = 256
_KNOTS = [float(v) for v in np.linspace(-1.0, 1.0, 10).astype(np.float32)]


def _sc_spline(x, atab):
    mesh = plsc.VectorSubcoreMesh(core_axis_name="c", subcore_axis_name="s")

    @functools.partial(
        pl.kernel,
        out_type=jax.ShapeDtypeStruct((_K,), jnp.float32),
        mesh=mesh,
        scratch_types=[
            pltpu.VMEM((_CH,), jnp.float32),
            pltpu.VMEM((_CH,), jnp.float32),
            pltpu.VMEM((_CH,), jnp.float32),
            pltpu.VMEM((_CH,), jnp.float32),
            pltpu.VMEM((_L,), jnp.float32),
            pltpu.VMEM((_L,), jnp.float32),
            pltpu.VMEM((_L,), jnp.float32),
            pltpu.VMEM((_L,), jnp.float32),
            pltpu.SemaphoreType.DMA,
            pltpu.SemaphoreType.DMA,
            pltpu.SemaphoreType.DMA,
            pltpu.SemaphoreType.DMA,
        ],
        compiler_params=pltpu.CompilerParams(needs_layout_passes=False),
    )
    def run(x_hbm, a_hbm, o_hbm, xb0, xb1, ob0, ob1, t0, t1, t2, t3,
            si0, si1, so0, so1):
        wid = lax.axis_index("s") * 2 + lax.axis_index("c")
        base = wid * _PER_W
        tabs = (t0, t1, t2, t3)
        for j in range(4):
            pltpu.sync_copy(a_hbm.at[j], tabs[j])

        xbufs, obufs = (xb0, xb1), (ob0, ob1)
        sins, souts = (si0, si1), (so0, so1)

        def compute(xbuf, obuf):
            @plsc.parallel_loop(0, _CH, _L, unroll=8)
            def body(s):
                xv = xbuf[pl.ds(s, _L)]
                xc = jnp.minimum(jnp.maximum(xv, -1.0), 1.0)
                f = (xc + 1.0) * 4.5
                idx = jnp.minimum(f.astype(jnp.int32), 8)
                a0 = plsc.load_gather(t0, [idx])
                a1 = plsc.load_gather(t1, [idx])
                a2 = plsc.load_gather(t2, [idx])
                a3 = plsc.load_gather(t3, [idx])
                obuf[pl.ds(s, _L)] = a0 + xc * (a1 + xc * (a2 + xc * a3))

        def off(g):
            return pl.multiple_of(base + g * _CH, 8)

        in_d = {0: pltpu.async_copy(x_hbm.at[pl.ds(off(0), _CH)], xb0, si0)}
        out_d = {}
        for g in range(_CHUNKS):
            cur = g % 2
            if g + 1 < _CHUNKS:
                in_d[g + 1] = pltpu.async_copy(
                    x_hbm.at[pl.ds(off(g + 1), _CH)],
                    xbufs[(g + 1) % 2], sins[(g + 1) % 2])
            in_d[g].wait()
            if g >= 2:
                out_d[g - 2].wait()
            compute(xbufs[cur], obufs[cur])
            out_d[g] = pltpu.async_copy(
                obufs[cur], o_hbm.at[pl.ds(off(g), _CH)], souts[cur])
        out_d[_CHUNKS - 2].wait()
        out_d[_CHUNKS - 1].wait()

    return run(x, atab)


def _tc_spline(x2d, atab):
    rows = _ROWS - _K_ROWS

    def body(a_ref, x_ref, o_ref):
        xc = jnp.minimum(jnp.maximum(x_ref[...], -1.0), 1.0)
        masks = [xc > _KNOTS[k] for k in range(1, 9)]
        coeffs = []
        for j in range(4):
            aj = jnp.full_like(xc, a_ref[0, j])
            for k in range(1, 9):
                aj = jnp.where(masks[k - 1], a_ref[k, j], aj)
            coeffs.append(aj)
        a0, a1, a2, a3 = coeffs
        o_ref[...] = a0 + xc * (a1 + xc * (a2 + xc * a3))

    return pl.pallas_call(
        body,
        grid=(rows // _BM,),
        in_specs=[
            pl.BlockSpec(memory_space=pltpu.SMEM),
            pl.BlockSpec((_BM, 1024), lambda i: (i + _K_ROWS // _BM, 0)),
        ],
        out_specs=pl.BlockSpec((_BM, 1024), lambda i: (i, 0)),
        out_shape=jax.ShapeDtypeStruct((rows, 1024), jnp.float32),
    )(atab, x2d)


def kernel(x, coefficients):
    knots = jnp.linspace(-1.0, 1.0, 10).astype(jnp.float32)
    lo, hi = knots[:-1], knots[1:]
    m = 1.0 / (hi - lo)
    b = -lo * m
    c0, c1, c2, c3 = (coefficients[:, i] for i in range(4))
    a0 = c0 + b * (c1 + b * (c2 + b * c3))
    a1 = m * (c1 + b * (2.0 * c2 + 3.0 * c3 * b))
    a2 = m * m * (c2 + 3.0 * c3 * b)
    a3 = m * m * m * c3
    atab_tc = jnp.stack([a0, a1, a2, a3], axis=-1)
    atab_sc = jnp.pad(atab_tc.T, ((0, 0), (0, _L - 9)))
    sc_out = _sc_spline(x, atab_sc)
    tc_out = _tc_spline(x.reshape(_ROWS, 1024), atab_tc)
    return jnp.concatenate([sc_out, tc_out.reshape(-1)])


# speedup vs baseline: 2.8101x; 2.8101x over previous
import functools

import jax
import jax.numpy as jnp
import numpy as np
from jax import lax
from jax.experimental import pallas as pl
from jax.experimental.pallas import tpu as pltpu
from jax.experimental.pallas import tpu_sc as plsc

_N = 16777216
_L = 16
_NW = 32
_CH = 16384
_K = 7340032
_PER_W = _K // _NW
_CHUNKS = _PER_W // _CH
_TC_BLK = 524288
_KNOTS = [float(v) for v in np.linspace(-1.0, 1.0, 10).astype(np.float32)]


def _sc_spline(x, atab):
    mesh = plsc.VectorSubcoreMesh(core_axis_name="c", subcore_axis_name="s")

    @functools.partial(
        pl.kernel,
        out_type=jax.ShapeDtypeStruct((_K,), jnp.float32),
        mesh=mesh,
        scratch_types=[
            pltpu.VMEM((_CH,), jnp.float32),
            pltpu.VMEM((_CH,), jnp.float32),
            pltpu.VMEM((_CH,), jnp.float32),
            pltpu.VMEM((_CH,), jnp.float32),
            pltpu.VMEM((_L,), jnp.float32),
            pltpu.VMEM((_L,), jnp.float32),
            pltpu.VMEM((_L,), jnp.float32),
            pltpu.VMEM((_L,), jnp.float32),
            pltpu.SemaphoreType.DMA,
            pltpu.SemaphoreType.DMA,
            pltpu.SemaphoreType.DMA,
            pltpu.SemaphoreType.DMA,
        ],
        compiler_params=pltpu.CompilerParams(needs_layout_passes=False),
    )
    def run(x_hbm, a_hbm, o_hbm, xb0, xb1, ob0, ob1, t0, t1, t2, t3,
            si0, si1, so0, so1):
        wid = lax.axis_index("s") * 2 + lax.axis_index("c")
        base = wid * _PER_W
        tabs = (t0, t1, t2, t3)
        for j in range(4):
            pltpu.sync_copy(a_hbm.at[j], tabs[j])

        xbufs, obufs = (xb0, xb1), (ob0, ob1)
        sins, souts = (si0, si1), (so0, so1)

        def compute(xbuf, obuf):
            @plsc.parallel_loop(0, _CH, _L, unroll=8)
            def body(s):
                xv = xbuf[pl.ds(s, _L)]
                xc = jnp.minimum(jnp.maximum(xv, -1.0), 1.0)
                f = (xc + 1.0) * 4.5
                idx = jnp.minimum(f.astype(jnp.int32), 8)
                a0 = plsc.load_gather(t0, [idx])
                a1 = plsc.load_gather(t1, [idx])
                a2 = plsc.load_gather(t2, [idx])
                a3 = plsc.load_gather(t3, [idx])
                obuf[pl.ds(s, _L)] = a0 + xc * (a1 + xc * (a2 + xc * a3))

        def off(g):
            return pl.multiple_of(base + g * _CH, 8)

        in_d = {0: pltpu.async_copy(x_hbm.at[pl.ds(off(0), _CH)], xb0, si0)}
        out_d = {}
        for g in range(_CHUNKS):
            cur = g % 2
            if g + 1 < _CHUNKS:
                in_d[g + 1] = pltpu.async_copy(
                    x_hbm.at[pl.ds(off(g + 1), _CH)],
                    xbufs[(g + 1) % 2], sins[(g + 1) % 2])
            in_d[g].wait()
            if g >= 2:
                out_d[g - 2].wait()
            compute(xbufs[cur], obufs[cur])
            out_d[g] = pltpu.async_copy(
                obufs[cur], o_hbm.at[pl.ds(off(g), _CH)], souts[cur])
        out_d[_CHUNKS - 2].wait()
        out_d[_CHUNKS - 1].wait()

    return run(x, atab)


def _tc_spline(x, atab):
    def body(a_ref, x_ref, o_ref):
        xc = jnp.minimum(jnp.maximum(x_ref[...], -1.0), 1.0)
        masks = [xc > _KNOTS[k] for k in range(1, 9)]
        coeffs = []
        for j in range(4):
            aj = jnp.full_like(xc, a_ref[0, j])
            for k in range(1, 9):
                aj = jnp.where(masks[k - 1], a_ref[k, j], aj)
            coeffs.append(aj)
        a0, a1, a2, a3 = coeffs
        o_ref[...] = a0 + xc * (a1 + xc * (a2 + xc * a3))

    return pl.pallas_call(
        body,
        grid=((_N - _K) // _TC_BLK,),
        in_specs=[
            pl.BlockSpec(memory_space=pltpu.SMEM),
            pl.BlockSpec((_TC_BLK,), lambda i: (i + _K // _TC_BLK,)),
        ],
        out_specs=pl.BlockSpec((_TC_BLK,), lambda i: (i + _K // _TC_BLK,)),
        out_shape=jax.ShapeDtypeStruct((_N,), jnp.float32),
    )(atab, x)


def kernel(x, coefficients):
    knots = jnp.linspace(-1.0, 1.0, 10).astype(jnp.float32)
    lo, hi = knots[:-1], knots[1:]
    m = 1.0 / (hi - lo)
    b = -lo * m
    c0, c1, c2, c3 = (coefficients[:, i] for i in range(4))
    a0 = c0 + b * (c1 + b * (c2 + b * c3))
    a1 = m * (c1 + b * (2.0 * c2 + 3.0 * c3 * b))
    a2 = m * m * (c2 + 3.0 * c3 * b)
    a3 = m * m * m * c3
    atab_tc = jnp.stack([a0, a1, a2, a3], axis=-1)
    atab_sc = jnp.pad(atab_tc.T, ((0, 0), (0, _L - 9)))
    sc_out = _sc_spline(x, atab_sc)
    tc_out = _tc_spline(x, atab_tc)
    return lax.dynamic_update_slice(tc_out, sc_out, (0,))
